# baseline (speedup 1.0000x reference)
import jax
import jax.numpy as jnp
from jax import lax
from jax.experimental import pallas as pl
from jax.experimental.pallas import tpu as pltpu
from jax.experimental.pallas import tpu_sc as plsc

NC = 2
NS = 16
L = 16
UNROLL = 4


def _make_body(V, B, CB):
    NW = NC * NS
    BW = B // NW
    NCHUNK = BW // CB

    def body(votes_hbm, w_hbm, out_hbm, chunk_a, chunk_b, out_a, out_b,
             w_v, sem_w, sems_in, sems_out):
        wid = lax.axis_index("s") * NC + lax.axis_index("c")
        base = wid * BW
        pltpu.async_copy(w_hbm, w_v, sem_w).wait()
        wlo = w_v[pl.ds(0, L)]
        whi = w_v[pl.ds(L, L)]
        ws = [wlo[v] if v < L else whi[v - L] for v in range(V)]
        total = ws[0]
        for v in range(1, V):
            total = total + ws[v]

        chunks = [chunk_a, chunk_b]
        outs = [out_a, out_b]

        def start_in(c, buf):
            col0 = base + c * CB
            return pltpu.async_copy(
                votes_hbm.at[:, pl.ds(col0, CB)], chunks[buf],
                sems_in.at[buf])

        in_copies = [start_in(0, 0), start_in(1, 1)]
        out_copies = [None, None]

        for c in range(NCHUNK):
            buf = c % 2
            in_copies[buf].wait()
            chunk_v, out_v = chunks[buf], outs[buf]
            if out_copies[buf] is not None:
                out_copies[buf].wait()

            def group_body(g, carry, chunk_v=chunk_v, out_v=out_v):
                for u in range(UNROLL):
                    sl = pl.ds((g * UNROLL + u) * L, L)
                    acc = ws[0] * chunk_v[0, sl].astype(jnp.float32)
                    for v in range(1, V):
                        acc = acc + ws[v] * chunk_v[v, sl].astype(jnp.float32)
                    out_v[sl] = jnp.where(
                        acc + acc > total, 1, 0).astype(jnp.int32)
                return carry

            lax.fori_loop(0, CB // (L * UNROLL), group_body, 0)

            if c + 2 < NCHUNK:
                in_copies[buf] = start_in(c + 2, buf)
            col0 = base + c * CB
            out_copies[buf] = pltpu.async_copy(
                out_v, out_hbm.at[pl.ds(col0, CB)], sems_out.at[buf])

        for oc in out_copies:
            if oc is not None:
                oc.wait()

    return body


def kernel(votes, vote_weights):
    V, B = votes.shape
    CB = 1024
    f = pl.kernel(
        _make_body(V, B, CB),
        out_type=jax.ShapeDtypeStruct((B,), jnp.int32),
        mesh=plsc.VectorSubcoreMesh(
            core_axis_name="c", subcore_axis_name="s",
            num_cores=NC, num_subcores=NS,
        ),
        scratch_types=[
            pltpu.VMEM((V, CB), jnp.int32),
            pltpu.VMEM((V, CB), jnp.int32),
            pltpu.VMEM((CB,), jnp.int32),
            pltpu.VMEM((CB,), jnp.int32),
            pltpu.VMEM((2 * L,), jnp.float32),
            pltpu.SemaphoreType.DMA,
            pltpu.SemaphoreType.DMA((2,)),
            pltpu.SemaphoreType.DMA((2,)),
        ],
    )
    w_pad = jnp.zeros((2 * L,), jnp.float32).at[:V].set(
        vote_weights.astype(jnp.float32))
    return f(votes, w_pad)


# speedup vs baseline: 1.2328x; 1.2328x over previous
import jax
import jax.numpy as jnp
from jax.experimental import pallas as pl
from jax.experimental.pallas import tpu as pltpu


def _tc_body(votes_ref, w_ref, out_ref):
    w = w_ref[...]
    total = jnp.sum(w)
    counts = jnp.sum(w * votes_ref[...].astype(jnp.float32), axis=0)
    out_ref[...] = jnp.where(counts + counts > total, 1, 0).astype(jnp.int32)


def kernel(votes, vote_weights):
    V, B = votes.shape
    NB = 4096
    grid = (B // NB,)
    w2 = vote_weights.astype(jnp.float32).reshape(V, 1)
    return pl.pallas_call(
        _tc_body,
        grid=grid,
        in_specs=[
            pl.BlockSpec((V, NB), lambda i: (0, i)),
            pl.BlockSpec((V, 1), lambda i: (0, 0)),
        ],
        out_specs=pl.BlockSpec((NB,), lambda i: (i,)),
        out_shape=jax.ShapeDtypeStruct((B,), jnp.int32),
    )(votes, w2)


# speedup vs baseline: 2.5153x; 2.0403x over previous
import jax
import jax.numpy as jnp
from jax.experimental import pallas as pl
from jax.experimental.pallas import tpu as pltpu


def _tc_body(votes_ref, w_ref, out_ref):
    w = w_ref[...]
    total = jnp.sum(w)
    counts = jnp.sum(w * votes_ref[...].astype(jnp.float32), axis=0)
    out_ref[...] = jnp.where(counts + counts > total, 1, 0).astype(jnp.int32)


def kernel(votes, vote_weights):
    V, B = votes.shape
    NB = 16384
    grid = (B // NB,)
    w2 = vote_weights.astype(jnp.float32).reshape(V, 1)
    return pl.pallas_call(
        _tc_body,
        grid=grid,
        in_specs=[
            pl.BlockSpec((V, NB), lambda i: (0, i)),
            pl.BlockSpec((V, 1), lambda i: (0, 0)),
        ],
        out_specs=pl.BlockSpec((NB,), lambda i: (i,)),
        out_shape=jax.ShapeDtypeStruct((B,), jnp.int32),
    )(votes, w2)


# speedup vs baseline: 2.9901x; 1.1888x over previous
import jax
import jax.numpy as jnp
from jax.experimental import pallas as pl
from jax.experimental.pallas import tpu as pltpu


def _tc_body(votes_ref, w_ref, out_ref):
    w = w_ref[...]
    total = jnp.sum(w)
    counts = jnp.sum(w * votes_ref[...].astype(jnp.float32), axis=0)
    out_ref[...] = jnp.where(counts + counts > total, 1, 0).astype(jnp.int32)


def kernel(votes, vote_weights):
    V, B = votes.shape
    NB = 32768
    grid = (B // NB,)
    w2 = vote_weights.astype(jnp.float32).reshape(V, 1)
    return pl.pallas_call(
        _tc_body,
        grid=grid,
        in_specs=[
            pl.BlockSpec((V, NB), lambda i: (0, i)),
            pl.BlockSpec((V, 1), lambda i: (0, 0)),
        ],
        out_specs=pl.BlockSpec((NB,), lambda i: (i,)),
        out_shape=jax.ShapeDtypeStruct((B,), jnp.int32),
    )(votes, w2)


# speedup vs baseline: 3.1147x; 1.0417x over previous
import jax
import jax.numpy as jnp
from jax.experimental import pallas as pl
from jax.experimental.pallas import tpu as pltpu


def _tc_body(votes_ref, w_ref, out_ref):
    w = w_ref[...]
    total = jnp.sum(w)
    counts = jnp.sum(w * votes_ref[...].astype(jnp.float32), axis=0)
    out_ref[...] = jnp.where(counts + counts > total, 1, 0).astype(jnp.int32)


def kernel(votes, vote_weights):
    V, B = votes.shape
    NB = 65536
    grid = (B // NB,)
    w2 = vote_weights.astype(jnp.float32).reshape(V, 1)
    return pl.pallas_call(
        _tc_body,
        grid=grid,
        in_specs=[
            pl.BlockSpec((V, NB), lambda i: (0, i)),
            pl.BlockSpec((V, 1), lambda i: (0, 0)),
        ],
        out_specs=pl.BlockSpec((NB,), lambda i: (i,)),
        out_shape=jax.ShapeDtypeStruct((B,), jnp.int32),
    )(votes, w2)
